# baseline (speedup 1.0000x reference)
import jax
import jax.numpy as jnp
from jax.experimental import pallas as pl


_BLOCK_ROWS = 2048


def _body(x_ref, hp_ref, out_ref):
    x = x_ref[...]
    k = hp_ref[0, :]
    m = hp_ref[1, :]
    d = x - m[None, :]
    out_ref[...] = jnp.sum(k[None, :] * d * d, axis=1, keepdims=True) * 0.5


def kernel(in_feat, harmonic_parameters):
    n, f = in_feat.shape
    grid = (n // _BLOCK_ROWS,)
    return pl.pallas_call(
        _body,
        grid=grid,
        in_specs=[
            pl.BlockSpec((_BLOCK_ROWS, f), lambda i: (i, 0)),
            pl.BlockSpec((2, f), lambda i: (0, 0)),
        ],
        out_specs=pl.BlockSpec((_BLOCK_ROWS, 1), lambda i: (i, 0)),
        out_shape=jax.ShapeDtypeStruct((n, 1), jnp.float32),
    )(in_feat, harmonic_parameters)


# speedup vs baseline: 1.0072x; 1.0072x over previous
import jax
import jax.numpy as jnp
from jax.experimental import pallas as pl


_BLOCK_ROWS = 2048


def _body(x_ref, hp_ref, out_ref):
    x = x_ref[...]
    k = hp_ref[0, :]
    m = hp_ref[1, :]
    km = k * m
    c = 0.5 * jnp.sum(km * m)
    t = x * (0.5 * k[None, :] * x - km[None, :])
    ones = jnp.ones((x.shape[1], 1), dtype=jnp.float32)
    out_ref[...] = (
        jax.lax.dot_general(
            t, ones, (((1,), (0,)), ((), ())), preferred_element_type=jnp.float32
        )
        + c
    )


def kernel(in_feat, harmonic_parameters):
    n, f = in_feat.shape
    grid = (n // _BLOCK_ROWS,)
    return pl.pallas_call(
        _body,
        grid=grid,
        in_specs=[
            pl.BlockSpec((_BLOCK_ROWS, f), lambda i: (i, 0)),
            pl.BlockSpec((2, f), lambda i: (0, 0)),
        ],
        out_specs=pl.BlockSpec((_BLOCK_ROWS, 1), lambda i: (i, 0)),
        out_shape=jax.ShapeDtypeStruct((n, 1), jnp.float32),
    )(in_feat, harmonic_parameters)
